# baseline (device time: 380223 ns/iter reference)
import jax
import jax.numpy as jnp
from jax import lax
from jax.experimental import pallas as pl
from jax.experimental.pallas import tpu as pltpu

P = 16


def kernel(x, w_mat, scale_x, scale_w):
    m_glob, k_per = x.shape
    _, n = w_mat.shape
    m_chunk = m_glob // P

    def body(x_ref, w_ref, sx_ref, sw_ref, out_ref,
             s_ref, comm_ref, send_sems, recv_sems, credit_sems):
        my = lax.axis_index("i")
        left = lax.rem(my + P - 1, P)
        right = lax.rem(my + 1, P)

        def local_chunk(c):
            xs = x_ref[pl.ds(c * m_chunk, m_chunk), :]
            return lax.dot_general(
                xs, w_ref[:, :], (((1,), (0,)), ((), ())),
                preferred_element_type=jnp.float32)

        barrier = pltpu.get_barrier_semaphore()
        for nbr in (left, right):
            pl.semaphore_signal(barrier, inc=1, device_id=(nbr,),
                                device_id_type=pl.DeviceIdType.MESH)
        pl.semaphore_wait(barrier, 2)

        s_ref[0] = local_chunk(lax.rem(my + P - 1, P))

        for h in range(P - 1):
            slot = h % 2
            if h >= 2:
                pl.semaphore_wait(credit_sems.at[slot], 1)
            rdma = pltpu.make_async_remote_copy(
                src_ref=s_ref.at[slot],
                dst_ref=comm_ref.at[slot],
                send_sem=send_sems.at[slot],
                recv_sem=recv_sems.at[slot],
                device_id=(right,),
                device_id_type=pl.DeviceIdType.MESH,
            )
            rdma.start()
            rdma.wait()

            c = lax.rem(my + 2 * P - h - 2, P)
            val = comm_ref[slot] + local_chunk(c)
            if h < P - 2:
                s_ref[(h + 1) % 2] = val
            else:
                scale = sx_ref[0] * sw_ref[0]
                out_ref[:, :] = jnp.maximum(val * scale, 0.0)
            if h <= P - 4:
                pl.semaphore_signal(credit_sems.at[slot], inc=1,
                                    device_id=(left,),
                                    device_id_type=pl.DeviceIdType.MESH)

    return pl.pallas_call(
        body,
        out_shape=jax.ShapeDtypeStruct((m_chunk, n), jnp.float32),
        in_specs=[
            pl.BlockSpec(memory_space=pltpu.VMEM),
            pl.BlockSpec(memory_space=pltpu.VMEM),
            pl.BlockSpec(memory_space=pltpu.SMEM),
            pl.BlockSpec(memory_space=pltpu.SMEM),
        ],
        out_specs=pl.BlockSpec(memory_space=pltpu.VMEM),
        scratch_shapes=[
            pltpu.VMEM((2, m_chunk, n), jnp.float32),
            pltpu.VMEM((2, m_chunk, n), jnp.float32),
            pltpu.SemaphoreType.DMA((2,)),
            pltpu.SemaphoreType.DMA((2,)),
            pltpu.SemaphoreType.REGULAR((2,)),
        ],
        compiler_params=pltpu.CompilerParams(collective_id=0),
    )(x, w_mat, scale_x, scale_w)


# device time: 128953 ns/iter; 2.9485x vs baseline; 2.9485x over previous
import jax
import jax.numpy as jnp
from jax import lax
from jax.experimental import pallas as pl
from jax.experimental.pallas import tpu as pltpu

P = 16
WIRE_DTYPE = jnp.bfloat16

RING = [0, 3, 7, 11, 15, 14, 10, 6, 2, 1, 5, 9, 13, 12, 8, 4]
POS = [0] * P
for _r, _l in enumerate(RING):
    POS[_l] = _r
NEXT = [RING[(POS[l] + 1) % P] for l in range(P)]
PREV = [RING[(POS[l] - 1) % P] for l in range(P)]


def kernel(x, w_mat, scale_x, scale_w):
    m_glob, k_per = x.shape
    _, n = w_mat.shape
    m_chunk = m_glob // P
    nh = n // 2

    my = lax.axis_index("i")
    pos = jnp.asarray(POS, jnp.int32)[my].reshape(1)
    nxt = jnp.asarray(NEXT, jnp.int32)[my].reshape(1)
    prv = jnp.asarray(PREV, jnp.int32)[my].reshape(1)
    ring = jnp.asarray(RING, jnp.int32)

    def body(x_ref, w_ref, sx_ref, sw_ref, pos_ref, nxt_ref, prv_ref,
             ring_ref, out_ref, s_cw, s_ccw, comm_cw, comm_ccw,
             send_sems, recv_sems, credit_sems):
        pos = pos_ref[0]
        nxt = nxt_ref[0]
        prv = prv_ref[0]

        def chunk_cw(c):
            xs = x_ref[pl.ds(c * m_chunk, m_chunk), :]
            return lax.dot_general(
                xs, w_ref[:, 0:nh], (((1,), (0,)), ((), ())),
                preferred_element_type=jnp.float32)

        def chunk_ccw(c):
            xs = x_ref[pl.ds(c * m_chunk, m_chunk), :]
            return lax.dot_general(
                xs, w_ref[:, nh:n], (((1,), (0,)), ((), ())),
                preferred_element_type=jnp.float32)

        barrier = pltpu.get_barrier_semaphore()
        for nbr in (prv, nxt):
            pl.semaphore_signal(barrier, inc=1, device_id=(nbr,),
                                device_id_type=pl.DeviceIdType.MESH)
        pl.semaphore_wait(barrier, 2)

        s_cw[0] = chunk_cw(ring_ref[lax.rem(pos + P - 1, P)]).astype(WIRE_DTYPE)
        s_ccw[0] = chunk_ccw(ring_ref[lax.rem(pos + 1, P)]).astype(WIRE_DTYPE)

        for h in range(P - 1):
            slot = h % 2
            if h >= 2:
                pl.semaphore_wait(credit_sems.at[0, slot], 1)
                pl.semaphore_wait(credit_sems.at[1, slot], 1)
            rdma_cw = pltpu.make_async_remote_copy(
                src_ref=s_cw.at[slot],
                dst_ref=comm_cw.at[slot],
                send_sem=send_sems.at[0, slot],
                recv_sem=recv_sems.at[0, slot],
                device_id=(nxt,),
                device_id_type=pl.DeviceIdType.MESH,
            )
            rdma_ccw = pltpu.make_async_remote_copy(
                src_ref=s_ccw.at[slot],
                dst_ref=comm_ccw.at[slot],
                send_sem=send_sems.at[1, slot],
                recv_sem=recv_sems.at[1, slot],
                device_id=(prv,),
                device_id_type=pl.DeviceIdType.MESH,
            )
            rdma_cw.start()
            rdma_ccw.start()

            c_cw = ring_ref[lax.rem(pos + 2 * P - h - 2, P)]
            c_ccw = ring_ref[lax.rem(pos + h + 2, P)]
            l_cw = chunk_cw(c_cw)
            l_ccw = chunk_ccw(c_ccw)

            rdma_cw.wait()
            rdma_ccw.wait()

            val_cw = comm_cw[slot].astype(jnp.float32) + l_cw
            val_ccw = comm_ccw[slot].astype(jnp.float32) + l_ccw
            if h < P - 2:
                s_cw[(h + 1) % 2] = val_cw.astype(WIRE_DTYPE)
                s_ccw[(h + 1) % 2] = val_ccw.astype(WIRE_DTYPE)
            else:
                scale = sx_ref[0] * sw_ref[0]
                out_ref[:, 0:nh] = jnp.maximum(val_cw * scale, 0.0)
                out_ref[:, nh:n] = jnp.maximum(val_ccw * scale, 0.0)
            if h <= P - 4:
                pl.semaphore_signal(credit_sems.at[0, slot], inc=1,
                                    device_id=(prv,),
                                    device_id_type=pl.DeviceIdType.MESH)
                pl.semaphore_signal(credit_sems.at[1, slot], inc=1,
                                    device_id=(nxt,),
                                    device_id_type=pl.DeviceIdType.MESH)

    return pl.pallas_call(
        body,
        out_shape=jax.ShapeDtypeStruct((m_chunk, n), jnp.float32),
        in_specs=[
            pl.BlockSpec(memory_space=pltpu.VMEM),
            pl.BlockSpec(memory_space=pltpu.VMEM),
            pl.BlockSpec(memory_space=pltpu.SMEM),
            pl.BlockSpec(memory_space=pltpu.SMEM),
            pl.BlockSpec(memory_space=pltpu.SMEM),
            pl.BlockSpec(memory_space=pltpu.SMEM),
            pl.BlockSpec(memory_space=pltpu.SMEM),
            pl.BlockSpec(memory_space=pltpu.SMEM),
        ],
        out_specs=pl.BlockSpec(memory_space=pltpu.VMEM),
        scratch_shapes=[
            pltpu.VMEM((2, m_chunk, nh), WIRE_DTYPE),
            pltpu.VMEM((2, m_chunk, nh), WIRE_DTYPE),
            pltpu.VMEM((2, m_chunk, nh), WIRE_DTYPE),
            pltpu.VMEM((2, m_chunk, nh), WIRE_DTYPE),
            pltpu.SemaphoreType.DMA((2, 2)),
            pltpu.SemaphoreType.DMA((2, 2)),
            pltpu.SemaphoreType.REGULAR((2, 2)),
        ],
        compiler_params=pltpu.CompilerParams(collective_id=0),
    )(x, w_mat, scale_x, scale_w, pos, nxt, prv, ring)


# device time: 101834 ns/iter; 3.7338x vs baseline; 1.2663x over previous
import jax
import jax.numpy as jnp
from jax import lax
from jax.experimental import pallas as pl
from jax.experimental.pallas import tpu as pltpu

P = 16
SUB = 4
WIRE_DTYPE = jnp.bfloat16

RING = [0, 3, 7, 11, 15, 14, 10, 6, 2, 1, 5, 9, 13, 12, 8, 4]
POS = [0] * P
for _r, _l in enumerate(RING):
    POS[_l] = _r
NEXT = [RING[(POS[l] + 1) % P] for l in range(P)]
PREV = [RING[(POS[l] - 1) % P] for l in range(P)]


def kernel(x, w_mat, scale_x, scale_w):
    m_glob, k_per = x.shape
    _, n = w_mat.shape
    m_chunk = m_glob // P
    nh = n // 2
    ns = nh // SUB

    my = lax.axis_index("i")
    pos = jnp.asarray(POS, jnp.int32)[my].reshape(1)
    nxt = jnp.asarray(NEXT, jnp.int32)[my].reshape(1)
    prv = jnp.asarray(PREV, jnp.int32)[my].reshape(1)
    ring = jnp.asarray(RING, jnp.int32)

    def body(x_ref, w_ref, sx_ref, sw_ref, pos_ref, nxt_ref, prv_ref,
             ring_ref, out_ref, s_cw, s_ccw, comm_cw, comm_ccw,
             send_sems, recv_sems, credit_sems):
        pos = pos_ref[0]
        nxt = nxt_ref[0]
        prv = prv_ref[0]

        def chunk_gemm(c, col0, col1):
            xs = x_ref[pl.ds(c * m_chunk, m_chunk), :]
            return lax.dot_general(
                xs, w_ref[:, col0:col1], (((1,), (0,)), ((), ())),
                preferred_element_type=jnp.float32)

        def mk(d, slot, s, target):
            buf = s_cw if d == 0 else s_ccw
            com = comm_cw if d == 0 else comm_ccw
            return pltpu.make_async_remote_copy(
                src_ref=buf.at[slot, s],
                dst_ref=com.at[slot, s],
                send_sem=send_sems.at[d, slot, s],
                recv_sem=recv_sems.at[d, slot, s],
                device_id=(target,),
                device_id_type=pl.DeviceIdType.MESH,
            )

        barrier = pltpu.get_barrier_semaphore()
        for nbr in (prv, nxt):
            pl.semaphore_signal(barrier, inc=1, device_id=(nbr,),
                                device_id_type=pl.DeviceIdType.MESH)
        pl.semaphore_wait(barrier, 2)

        for h in range(P - 1):
            slot = h % 2
            pslot = (h - 1) % 2
            c_cw = ring_ref[lax.rem(pos + 2 * P - h - 1, P)]
            c_ccw = ring_ref[lax.rem(pos + h + 1, P)]
            l_cw = chunk_gemm(c_cw, 0, nh)
            l_ccw = chunk_gemm(c_ccw, nh, n)
            if h >= 2:
                pl.semaphore_wait(credit_sems.at[0, slot], 1)
                pl.semaphore_wait(credit_sems.at[1, slot], 1)
            for s in range(SUB):
                if h > 0:
                    mk(0, pslot, s, nxt).wait_recv()
                    mk(1, pslot, s, prv).wait_recv()
                    v_cw = comm_cw[pslot, s].astype(jnp.float32) \
                        + l_cw[:, s * ns:(s + 1) * ns]
                    v_ccw = comm_ccw[pslot, s].astype(jnp.float32) \
                        + l_ccw[:, s * ns:(s + 1) * ns]
                else:
                    v_cw = l_cw[:, s * ns:(s + 1) * ns]
                    v_ccw = l_ccw[:, s * ns:(s + 1) * ns]
                if h >= 2:
                    mk(0, slot, s, nxt).wait_send()
                    mk(1, slot, s, prv).wait_send()
                s_cw[slot, s] = v_cw.astype(WIRE_DTYPE)
                s_ccw[slot, s] = v_ccw.astype(WIRE_DTYPE)
                mk(0, slot, s, nxt).start()
                mk(1, slot, s, prv).start()
            if 1 <= h <= P - 3:
                pl.semaphore_signal(credit_sems.at[0, pslot], inc=1,
                                    device_id=(prv,),
                                    device_id_type=pl.DeviceIdType.MESH)
                pl.semaphore_signal(credit_sems.at[1, pslot], inc=1,
                                    device_id=(nxt,),
                                    device_id_type=pl.DeviceIdType.MESH)

        scale = sx_ref[0] * sw_ref[0]
        c_fin = ring_ref[pos]
        f_cw = chunk_gemm(c_fin, 0, nh)
        f_ccw = chunk_gemm(c_fin, nh, n)
        fslot = (P - 2) % 2
        for s in range(SUB):
            mk(0, fslot, s, nxt).wait_recv()
            mk(1, fslot, s, prv).wait_recv()
            v_cw = comm_cw[fslot, s].astype(jnp.float32) \
                + f_cw[:, s * ns:(s + 1) * ns]
            v_ccw = comm_ccw[fslot, s].astype(jnp.float32) \
                + f_ccw[:, s * ns:(s + 1) * ns]
            out_ref[:, s * ns:(s + 1) * ns] = jnp.maximum(v_cw * scale, 0.0)
            out_ref[:, nh + s * ns:nh + (s + 1) * ns] = \
                jnp.maximum(v_ccw * scale, 0.0)
        for hh in (P - 3, P - 2):
            for s in range(SUB):
                mk(0, hh % 2, s, nxt).wait_send()
                mk(1, hh % 2, s, prv).wait_send()

    return pl.pallas_call(
        body,
        out_shape=jax.ShapeDtypeStruct((m_chunk, n), jnp.float32),
        in_specs=[
            pl.BlockSpec(memory_space=pltpu.VMEM),
            pl.BlockSpec(memory_space=pltpu.VMEM),
            pl.BlockSpec(memory_space=pltpu.SMEM),
            pl.BlockSpec(memory_space=pltpu.SMEM),
            pl.BlockSpec(memory_space=pltpu.SMEM),
            pl.BlockSpec(memory_space=pltpu.SMEM),
            pl.BlockSpec(memory_space=pltpu.SMEM),
            pl.BlockSpec(memory_space=pltpu.SMEM),
        ],
        out_specs=pl.BlockSpec(memory_space=pltpu.VMEM),
        scratch_shapes=[
            pltpu.VMEM((2, SUB, m_chunk, ns), WIRE_DTYPE),
            pltpu.VMEM((2, SUB, m_chunk, ns), WIRE_DTYPE),
            pltpu.VMEM((2, SUB, m_chunk, ns), WIRE_DTYPE),
            pltpu.VMEM((2, SUB, m_chunk, ns), WIRE_DTYPE),
            pltpu.SemaphoreType.DMA((2, 2, SUB)),
            pltpu.SemaphoreType.DMA((2, 2, SUB)),
            pltpu.SemaphoreType.REGULAR((2, 2)),
        ],
        compiler_params=pltpu.CompilerParams(collective_id=0),
    )(x, w_mat, scale_x, scale_w, pos, nxt, prv, ring)
